# baseline (speedup 1.0000x reference)
import functools
import math

import jax
import jax.numpy as jnp
from jax import lax
from jax.experimental import pallas as pl
from jax.experimental.pallas import tpu as pltpu
from jax.experimental.pallas import tpu_sc as plsc

_INFO = plsc.get_sparse_core_info()
_NC = _INFO.num_cores
_NS = _INFO.num_subcores
_L = _INFO.num_lanes
_NW = _NC * _NS

_CH = 128


@functools.lru_cache(maxsize=None)
def _make_gather(B, D, num_objs):
    b_per_w = B // _NW
    n_ch = b_per_w // _CH
    mesh = plsc.VectorSubcoreMesh(core_axis_name="c", subcore_axis_name="s")

    @functools.partial(
        pl.kernel,
        mesh=mesh,
        out_type=jax.ShapeDtypeStruct((B, D), jnp.float32),
        compiler_params=pltpu.CompilerParams(use_tc_tiling_on_sc=False),
        scratch_types=[
            pltpu.VMEM((b_per_w,), jnp.int32),
            pltpu.VMEM((b_per_w,), jnp.int32),
            pltpu.VMEM((n_ch, _CH), jnp.int32),
            pltpu.VMEM((b_per_w, D), jnp.float32),
            pltpu.SemaphoreType.DMA,
        ],
    )
    def gather_kernel(l0_hbm, l1_hbm, table_hbm, out_hbm,
                      l0_v, l1_v, idx_v, rows_v, sem):
        wid = lax.axis_index("s") * _NC + lax.axis_index("c")
        base = wid * b_per_w
        pltpu.sync_copy(l0_hbm.at[pl.ds(base, b_per_w)], l0_v)
        pltpu.sync_copy(l1_hbm.at[pl.ds(base, b_per_w)], l1_v)
        for j in range(n_ch):
            for i in range(_CH // _L):
                off = j * _CH + i * _L
                a = l0_v[pl.ds(off, _L)]
                b = l1_v[pl.ds(off, _L)]
                idx_v[j, pl.ds(i * _L, _L)] = a * num_objs + b
        copies = [
            pltpu.async_copy(table_hbm.at[idx_v.at[j]],
                             rows_v.at[pl.ds(j * _CH, _CH)], sem)
            for j in range(n_ch)
        ]
        for c in copies:
            c.wait()
        pltpu.sync_copy(rows_v, out_hbm.at[pl.ds(base, b_per_w)])

    return gather_kernel


def kernel(labels, table, num_objs):
    B = labels.shape[0]
    D = table.shape[1]
    n = math.isqrt(table.shape[0])
    l0 = labels[:, 0]
    l1 = labels[:, 1]
    return _make_gather(B, D, n)(l0, l1, table)
